# baseline (speedup 1.0000x reference)
import functools

import jax
import jax.numpy as jnp
from jax import lax
from jax.experimental import pallas as pl
from jax.experimental.pallas import tpu as pltpu
from jax.experimental.pallas import tpu_sc as plsc

BATCH = 16384
N_FIELDS = 26
EMBED_DIM = 16

B = BATCH * N_FIELDS
NC, NS = 2, 16
NW = NC * NS
BPW = B // NW
CHUNK = 3328
NCHUNK = BPW // CHUNK

_mesh = plsc.VectorSubcoreMesh(core_axis_name="c", subcore_axis_name="s")


@functools.partial(
    pl.kernel,
    mesh=_mesh,
    out_type=jax.ShapeDtypeStruct((B, EMBED_DIM), jnp.float32),
    compiler_params=pltpu.CompilerParams(use_tc_tiling_on_sc=False),
    scratch_types=[
        pltpu.VMEM((CHUNK,), jnp.int32),
        pltpu.VMEM((CHUNK,), jnp.int32),
        pltpu.VMEM((CHUNK, EMBED_DIM), jnp.float32),
        pltpu.VMEM((CHUNK, EMBED_DIM), jnp.float32),
        pltpu.SemaphoreType.DMA,
        pltpu.SemaphoreType.DMA,
    ],
)
def _sc_gather(table_hbm, idx_hbm, out_hbm, idx0, idx1, rows0, rows1,
               sem0, sem1):
    wid = lax.axis_index("s") * NC + lax.axis_index("c")
    base = wid * BPW
    idx_bufs = (idx0, idx1)
    row_bufs = (rows0, rows1)
    sems = (sem0, sem1)

    pltpu.sync_copy(idx_hbm.at[pl.ds(base, CHUNK)], idx_bufs[0])
    copies = [None] * NCHUNK
    copies[0] = pltpu.async_copy(table_hbm.at[idx_bufs[0]], row_bufs[0],
                                 sems[0])
    for k in range(NCHUNK):
        if k + 1 < NCHUNK:
            nb = (k + 1) % 2
            pltpu.sync_copy(
                idx_hbm.at[pl.ds(base + (k + 1) * CHUNK, CHUNK)],
                idx_bufs[nb])
            copies[k + 1] = pltpu.async_copy(
                table_hbm.at[idx_bufs[nb]], row_bufs[nb], sems[nb])
        copies[k].wait()
        pltpu.sync_copy(row_bufs[k % 2],
                        out_hbm.at[pl.ds(base + k * CHUNK, CHUNK)])


@jax.jit
def kernel(indices, table):
    idx_flat = indices.reshape(-1).astype(jnp.int32)
    out = _sc_gather(table, idx_flat)
    return out.reshape(BATCH, N_FIELDS, EMBED_DIM)


# speedup vs baseline: 1.2815x; 1.2815x over previous
import functools

import jax
import jax.numpy as jnp
from jax import lax
from jax.experimental import pallas as pl
from jax.experimental.pallas import tpu as pltpu
from jax.experimental.pallas import tpu_sc as plsc

BATCH = 16384
N_FIELDS = 26
EMBED_DIM = 16
VOCAB = 1000000

B = BATCH * N_FIELDS
NC, NS = 2, 16
NW = NC * NS
NBLK_FULL = VOCAB // 128
TAIL_START = NBLK_FULL * 128
TAIL_N = VOCAB - TAIL_START
NBLOCKS = N_FIELDS * (BATCH // 128)
BLK_PER_W = NBLOCKS // NW
CHUNK_BLKS = 8
NCHUNK = BLK_PER_W // CHUNK_BLKS

_mesh = plsc.VectorSubcoreMesh(core_axis_name="c", subcore_axis_name="s")


def _iota16():
    return lax.iota(jnp.int32, 16)


_detile_params = pltpu.CompilerParams(use_tc_tiling_on_sc=True,
                                      needs_layout_passes=False)
_gather_params = pltpu.CompilerParams(use_tc_tiling_on_sc=False,
                                      needs_layout_passes=False)


@functools.partial(
    pl.kernel,
    mesh=_mesh,
    out_type=(
        jax.ShapeDtypeStruct((VOCAB * EMBED_DIM,), jnp.float32),
        jax.ShapeDtypeStruct((B,), jnp.int32),
    ),
    compiler_params=_detile_params,
    scratch_types=[
        pltpu.VMEM((EMBED_DIM, 128), jnp.float32),
        pltpu.VMEM((128 * EMBED_DIM,), jnp.float32),
        pltpu.VMEM((BATCH,), jnp.int32),
        pltpu.VMEM((EMBED_DIM, TAIL_N), jnp.float32),
        pltpu.VMEM((TAIL_N * EMBED_DIM,), jnp.float32),
    ],
)
def _detile(tt, it, tab_lin, idx_lin, vin, vout, idxrow, tailbuf, tailout):
    wid = lax.axis_index("s") * NC + lax.axis_index("c")
    i16 = _iota16()

    @pl.when(wid < N_FIELDS)
    def _():
        pltpu.sync_copy(it.at[wid], idxrow)
        pltpu.sync_copy(idxrow, idx_lin.at[pl.ds(wid * BATCH, BATCH)])

    n_t = jnp.where(wid < NBLK_FULL - (NBLK_FULL // NW) * NW, 1, 0) + NBLK_FULL // NW

    def blk_body(t, carry):
        tc = wid + NW * t
        pltpu.sync_copy(tt.at[:, pl.ds(tc * 128, 128)], vin)

        def tr_body(r, c2):
            v = plsc.load_gather(vin, [i16, jnp.full((16,), r, jnp.int32)])
            vout[pl.ds(r * EMBED_DIM, EMBED_DIM)] = v
            return c2

        lax.fori_loop(0, 128, tr_body, 0)
        pltpu.sync_copy(vout, tab_lin.at[pl.ds(tc * 128 * EMBED_DIM,
                                               128 * EMBED_DIM)])
        return carry

    lax.fori_loop(0, n_t, blk_body, 0)

    @pl.when(wid == NW - 1)
    def _():
        def te_body(e, c2):
            pltpu.sync_copy(tt.at[e, pl.ds(TAIL_START, TAIL_N)], tailbuf.at[e])
            return c2

        lax.fori_loop(0, EMBED_DIM, te_body, 0)

        def tr2_body(r, c2):
            v = plsc.load_gather(tailbuf, [i16, jnp.full((16,), r, jnp.int32)])
            tailout[pl.ds(r * EMBED_DIM, EMBED_DIM)] = v
            return c2

        lax.fori_loop(0, TAIL_N, tr2_body, 0)
        pltpu.sync_copy(tailout,
                        tab_lin.at[pl.ds(TAIL_START * EMBED_DIM,
                                         TAIL_N * EMBED_DIM)])


@functools.partial(
    pl.kernel,
    mesh=_mesh,
    out_type=jax.ShapeDtypeStruct((N_FIELDS, 2, BATCH // 128, 8, 128),
                                  jnp.float32),
    compiler_params=_gather_params,
    scratch_types=[
        pltpu.VMEM((CHUNK_BLKS * 128,), jnp.int32),
        pltpu.VMEM((CHUNK_BLKS * 128, EMBED_DIM), jnp.float32),
        pltpu.VMEM((2, CHUNK_BLKS, 8, 128), jnp.float32),
        pltpu.SemaphoreType.DMA,
    ],
)
def _gather(tab2d, idx_lin, out5, idxb, rows, outb, sem):
    wid = lax.axis_index("s") * NC + lax.axis_index("c")
    i16 = _iota16()

    def chunk_body(j, carry):
        g0 = wid * BLK_PER_W + CHUNK_BLKS * j
        f = g0 // (BATCH // 128)
        bt0 = g0 % (BATCH // 128)
        pltpu.sync_copy(idx_lin.at[pl.ds(g0 * 128, CHUNK_BLKS * 128)], idxb)
        pltpu.async_copy(tab2d.at[idxb], rows, sem).wait()

        def line_body(m, c2):
            k = m // EMBED_DIM
            e = m % EMBED_DIM
            g2 = e // 8
            e8 = e % 8
            for j2 in range(8):
                ridx = k * 128 + 16 * j2 + i16
                v = plsc.load_gather(rows, [ridx, jnp.full((16,), e, jnp.int32)])
                outb[g2, k, e8, pl.ds(16 * j2, 16)] = v
            return c2

        lax.fori_loop(0, CHUNK_BLKS * EMBED_DIM, line_body, 0)
        pltpu.sync_copy(outb.at[0], out5.at[f, 0, pl.ds(bt0, CHUNK_BLKS)])
        pltpu.sync_copy(outb.at[1], out5.at[f, 1, pl.ds(bt0, CHUNK_BLKS)])
        return carry

    lax.fori_loop(0, NCHUNK, chunk_body, 0)


@jax.jit
def kernel(indices, table):
    tab_lin, idx_lin = _detile(table.T, indices.T)
    out5 = _gather(tab_lin.reshape(VOCAB, EMBED_DIM), idx_lin)
    return out5.transpose(2, 4, 0, 1, 3).reshape(BATCH, N_FIELDS, EMBED_DIM)


# speedup vs baseline: 1.8404x; 1.4361x over previous
import functools

import jax
import jax.numpy as jnp
from jax import lax
from jax.experimental import pallas as pl
from jax.experimental.pallas import tpu as pltpu
from jax.experimental.pallas import tpu_sc as plsc

BATCH = 16384
N_FIELDS = 26
EMBED_DIM = 16
VOCAB = 1000000

B = BATCH * N_FIELDS
NC, NS = 2, 16
NW = NC * NS
C_SB = 512
N_SB_W = 61
SB_COVER = NW * N_SB_W * C_SB
N_LEFT = (VOCAB - SB_COVER) // 128
TAIL_START = SB_COVER + N_LEFT * 128
TAIL_N = VOCAB - TAIL_START
NBLOCKS = N_FIELDS * (BATCH // 128)
BLK_PER_W = NBLOCKS // NW
CHUNK_BLKS = 8
NCHUNK = BLK_PER_W // CHUNK_BLKS

_mesh = plsc.VectorSubcoreMesh(core_axis_name="c", subcore_axis_name="s")

_detile_params = pltpu.CompilerParams(use_tc_tiling_on_sc=True,
                                      needs_layout_passes=False)
_gather_params = pltpu.CompilerParams(use_tc_tiling_on_sc=False,
                                      needs_layout_passes=False)


def _iota16():
    return lax.iota(jnp.int32, 16)


@functools.partial(
    pl.kernel,
    mesh=_mesh,
    out_type=(
        jax.ShapeDtypeStruct((VOCAB * EMBED_DIM,), jnp.float32),
        jax.ShapeDtypeStruct((B,), jnp.int32),
    ),
    compiler_params=_detile_params,
    scratch_types=[
        pltpu.VMEM((EMBED_DIM, C_SB), jnp.float32),
        pltpu.VMEM((EMBED_DIM, C_SB), jnp.float32),
        pltpu.VMEM((C_SB * EMBED_DIM,), jnp.float32),
        pltpu.VMEM((C_SB * EMBED_DIM,), jnp.float32),
        pltpu.VMEM((BATCH,), jnp.int32),
        pltpu.VMEM((EMBED_DIM, TAIL_N), jnp.float32),
        pltpu.VMEM((TAIL_N * EMBED_DIM,), jnp.float32),
        pltpu.SemaphoreType.DMA,
        pltpu.SemaphoreType.DMA,
        pltpu.SemaphoreType.DMA,
        pltpu.SemaphoreType.DMA,
    ],
)
def _detile(tt, it, tab_lin, idx_lin, vin0, vin1, vout0, vout1, idxrow,
            tailbuf, tailout, isem0, isem1, osem0, osem1):
    wid = lax.axis_index("s") * NC + lax.axis_index("c")
    i16 = _iota16()
    vin_ = (vin0, vin1)
    vout_ = (vout0, vout1)
    isem_ = (isem0, isem1)
    osem_ = (osem0, osem1)

    @pl.when(wid < N_FIELDS)
    def _():
        pltpu.sync_copy(it.at[wid], idxrow)
        pltpu.sync_copy(idxrow, idx_lin.at[pl.ds(wid * BATCH, BATCH)])

    base_col = wid * N_SB_W * C_SB

    def in_desc(t, p):
        return pltpu.make_async_copy(
            tt.at[:, pl.ds(base_col + t * C_SB, C_SB)], vin_[p], isem_[p])

    def out_desc(t, p):
        return pltpu.make_async_copy(
            vout_[p],
            tab_lin.at[pl.ds((base_col + t * C_SB) * EMBED_DIM,
                             C_SB * EMBED_DIM)],
            osem_[p])

    def transpose_sb(p):
        def tr_body(m, c2):
            for jj in range(8):
                r = m * 8 + jj
                v = plsc.load_gather(vin_[p],
                                     [i16, jnp.full((16,), r, jnp.int32)])
                vout_[p][pl.ds(r * EMBED_DIM, EMBED_DIM)] = v
            return c2

        lax.fori_loop(0, C_SB // 8, tr_body, 0)

    in_desc(0, 0).start()
    in_desc(0, 0).wait()
    in_desc(1, 1).start()
    transpose_sb(0)
    out_desc(0, 0).start()
    in_desc(2, 0).start()
    in_desc(1, 1).wait()
    transpose_sb(1)
    out_desc(1, 1).start()
    in_desc(3, 1).start()

    def pair_body(u, c):
        t0 = 2 * u
        t1 = t0 + 1
        in_desc(t0, 0).wait()
        out_desc(t0 - 2, 0).wait()
        transpose_sb(0)
        out_desc(t0, 0).start()
        in_desc(t0 + 2, 0).start()
        in_desc(t1, 1).wait()
        out_desc(t1 - 2, 1).wait()
        transpose_sb(1)
        out_desc(t1, 1).start()

        @pl.when(t1 + 2 < N_SB_W)
        def _():
            in_desc(t1 + 2, 1).start()

        return c

    lax.fori_loop(1, (N_SB_W - 1) // 2, pair_body, 0)
    in_desc(N_SB_W - 1, 0).wait()
    out_desc(N_SB_W - 3, 0).wait()
    transpose_sb(0)
    out_desc(N_SB_W - 1, 0).start()
    out_desc(N_SB_W - 2, 1).wait()
    out_desc(N_SB_W - 1, 0).wait()

    @pl.when((wid >= 16) & (wid < 16 + N_LEFT))
    def _():
        c0 = SB_COVER + (wid - 16) * 128
        pltpu.sync_copy(tt.at[:, pl.ds(c0, 128)], vin_[0].at[:, pl.ds(0, 128)])

        def trl_body(m, c2):
            for jj in range(8):
                r = m * 8 + jj
                v = plsc.load_gather(vin_[0],
                                     [i16, jnp.full((16,), r, jnp.int32)])
                vout_[0][pl.ds(r * EMBED_DIM, EMBED_DIM)] = v
            return c2

        lax.fori_loop(0, 16, trl_body, 0)
        pltpu.sync_copy(vout_[0].at[pl.ds(0, 128 * EMBED_DIM)],
                        tab_lin.at[pl.ds(c0 * EMBED_DIM, 128 * EMBED_DIM)])

    @pl.when(wid == 20)
    def _():
        def te_body(e, c2):
            pltpu.sync_copy(tt.at[e, pl.ds(TAIL_START, TAIL_N)], tailbuf.at[e])
            return c2

        lax.fori_loop(0, EMBED_DIM, te_body, 0)

        def tr2_body(r, c2):
            v = plsc.load_gather(tailbuf, [i16, jnp.full((16,), r, jnp.int32)])
            tailout[pl.ds(r * EMBED_DIM, EMBED_DIM)] = v
            return c2

        lax.fori_loop(0, TAIL_N, tr2_body, 0)
        pltpu.sync_copy(tailout,
                        tab_lin.at[pl.ds(TAIL_START * EMBED_DIM,
                                         TAIL_N * EMBED_DIM)])


@functools.partial(
    pl.kernel,
    mesh=_mesh,
    out_type=jax.ShapeDtypeStruct((N_FIELDS, 2, BATCH // 128, 8, 128),
                                  jnp.float32),
    compiler_params=_gather_params,
    scratch_types=[
        pltpu.VMEM((BLK_PER_W * 128,), jnp.int32),
        pltpu.VMEM((CHUNK_BLKS * 128, EMBED_DIM), jnp.float32),
        pltpu.VMEM((CHUNK_BLKS * 128, EMBED_DIM), jnp.float32),
        pltpu.VMEM((2, CHUNK_BLKS, 8, 128), jnp.float32),
        pltpu.VMEM((2, CHUNK_BLKS, 8, 128), jnp.float32),
        pltpu.SemaphoreType.DMA,
        pltpu.SemaphoreType.DMA,
        pltpu.SemaphoreType.DMA,
        pltpu.SemaphoreType.DMA,
    ],
)
def _gather(tab2d, idx_lin, out5, idxall, rows0, rows1, outb0, outb1,
            gsem0, gsem1, wsem0, wsem1):
    wid = lax.axis_index("s") * NC + lax.axis_index("c")
    i16 = _iota16()
    rows_ = (rows0, rows1)
    outb_ = (outb0, outb1)
    gsem_ = (gsem0, gsem1)
    wsem_ = (wsem0, wsem1)

    base_g = wid * BLK_PER_W
    pltpu.sync_copy(idx_lin.at[pl.ds(base_g * 128, BLK_PER_W * 128)], idxall)

    def g_start(j, p):
        return pltpu.async_copy(
            tab2d.at[idxall.at[pl.ds(j * CHUNK_BLKS * 128, CHUNK_BLKS * 128)]],
            rows_[p], gsem_[p])

    cp = g_start(0, 0)
    prev_out = [None, None]
    for j in range(NCHUNK):
        p = j & 1
        nxt = g_start(j + 1, 1 - p) if j + 1 < NCHUNK else None
        cp.wait()
        if prev_out[p] is not None:
            prev_out[p][0].wait()
            prev_out[p][1].wait()
        g0 = base_g + CHUNK_BLKS * j
        f = g0 // (BATCH // 128)
        bt0 = g0 % (BATCH // 128)

        def line_body(m, c2, p=p):
            k = m // EMBED_DIM
            e = m % EMBED_DIM
            g2 = e // 8
            e8 = e % 8
            for j2 in range(8):
                ridx = k * 128 + 16 * j2 + i16
                v = plsc.load_gather(rows_[p],
                                     [ridx, jnp.full((16,), e, jnp.int32)])
                outb_[p][g2, k, e8, pl.ds(16 * j2, 16)] = v
            return c2

        lax.fori_loop(0, CHUNK_BLKS * EMBED_DIM, line_body, 0)
        o1 = pltpu.async_copy(outb_[p].at[0],
                              out5.at[f, 0, pl.ds(bt0, CHUNK_BLKS)], wsem_[p])
        o2 = pltpu.async_copy(outb_[p].at[1],
                              out5.at[f, 1, pl.ds(bt0, CHUNK_BLKS)], wsem_[p])
        prev_out[p] = (o1, o2)
        cp = nxt
    for p in (0, 1):
        if prev_out[p] is not None:
            prev_out[p][0].wait()
            prev_out[p][1].wait()


@jax.jit
def kernel(indices, table):
    tab_lin, idx_lin = _detile(table.T, indices.T)
    out5 = _gather(tab_lin.reshape(VOCAB, EMBED_DIM), idx_lin)
    return out5.transpose(2, 4, 0, 1, 3).reshape(BATCH, N_FIELDS, EMBED_DIM)


# speedup vs baseline: 2.9277x; 1.5908x over previous
import functools

import jax
import jax.numpy as jnp
from jax import lax
from jax.experimental import pallas as pl
from jax.experimental.pallas import tpu as pltpu
from jax.experimental.pallas import tpu_sc as plsc

BATCH = 16384
N_FIELDS = 26
EMBED_DIM = 16
VOCAB = 1000000

B = BATCH * N_FIELDS
NC, NS = 2, 16
NW = NC * NS
C_SB = 512
N_SB_W = 61
SB_COVER = NW * N_SB_W * C_SB
N_LEFT = (VOCAB - SB_COVER) // 128
TAIL_START = SB_COVER + N_LEFT * 128
TAIL_N = VOCAB - TAIL_START
NBLOCKS = N_FIELDS * (BATCH // 128)
BLK_PER_W = NBLOCKS // NW
CHUNK_BLKS = 8
NCHUNK = BLK_PER_W // CHUNK_BLKS

_mesh = plsc.VectorSubcoreMesh(core_axis_name="c", subcore_axis_name="s")

_detile_params = pltpu.CompilerParams(use_tc_tiling_on_sc=True,
                                      needs_layout_passes=False)
_gather_params = pltpu.CompilerParams(use_tc_tiling_on_sc=False,
                                      needs_layout_passes=False)


def _iota16():
    return lax.iota(jnp.int32, 16)


@functools.partial(
    pl.kernel,
    mesh=_mesh,
    out_type=(
        jax.ShapeDtypeStruct((VOCAB * EMBED_DIM,), jnp.float32),
        jax.ShapeDtypeStruct((B,), jnp.int32),
    ),
    compiler_params=_detile_params,
    scratch_types=[
        pltpu.VMEM((EMBED_DIM, C_SB), jnp.float32),
        pltpu.VMEM((EMBED_DIM, C_SB), jnp.float32),
        pltpu.VMEM((C_SB * EMBED_DIM,), jnp.float32),
        pltpu.VMEM((C_SB * EMBED_DIM,), jnp.float32),
        pltpu.VMEM((BATCH,), jnp.int32),
        pltpu.VMEM((EMBED_DIM, TAIL_N), jnp.float32),
        pltpu.VMEM((TAIL_N * EMBED_DIM,), jnp.float32),
        pltpu.SemaphoreType.DMA,
        pltpu.SemaphoreType.DMA,
        pltpu.SemaphoreType.DMA,
        pltpu.SemaphoreType.DMA,
    ],
)
def _detile(tt, it, tab_lin, idx_lin, vin0, vin1, vout0, vout1, idxrow,
            tailbuf, tailout, isem0, isem1, osem0, osem1):
    wid = lax.axis_index("s") * NC + lax.axis_index("c")
    i16 = _iota16()
    vin_ = (vin0, vin1)
    vout_ = (vout0, vout1)
    isem_ = (isem0, isem1)
    osem_ = (osem0, osem1)

    @pl.when(wid < N_FIELDS)
    def _():
        pltpu.sync_copy(it.at[wid], idxrow)
        pltpu.sync_copy(idxrow, idx_lin.at[pl.ds(wid * BATCH, BATCH)])

    base_col = wid * N_SB_W * C_SB

    def in_desc(t, p):
        return pltpu.make_async_copy(
            tt.at[:, pl.ds(base_col + t * C_SB, C_SB)], vin_[p], isem_[p])

    def out_desc(t, p):
        return pltpu.make_async_copy(
            vout_[p],
            tab_lin.at[pl.ds((base_col + t * C_SB) * EMBED_DIM,
                             C_SB * EMBED_DIM)],
            osem_[p])

    i16x16 = i16 * EMBED_DIM

    def transpose_sb(p):
        def e_body(e, c2):
            def c_body(q, idxv):
                for u in range(4):
                    v = vin_[p][e, pl.ds((q * 4 + u) * 16, 16)]
                    plsc.store_scatter(vout_[p], [idxv + (256 * u)], v)
                return idxv + 1024

            lax.fori_loop(0, C_SB // 64, c_body, i16x16 + e)
            return c2

        lax.fori_loop(0, EMBED_DIM, e_body, 0)

    in_desc(0, 0).start()
    in_desc(0, 0).wait()
    in_desc(1, 1).start()
    transpose_sb(0)
    out_desc(0, 0).start()
    in_desc(2, 0).start()
    in_desc(1, 1).wait()
    transpose_sb(1)
    out_desc(1, 1).start()
    in_desc(3, 1).start()

    def pair_body(u, c):
        t0 = 2 * u
        t1 = t0 + 1
        in_desc(t0, 0).wait()
        out_desc(t0 - 2, 0).wait()
        transpose_sb(0)
        out_desc(t0, 0).start()
        in_desc(t0 + 2, 0).start()
        in_desc(t1, 1).wait()
        out_desc(t1 - 2, 1).wait()
        transpose_sb(1)
        out_desc(t1, 1).start()

        @pl.when(t1 + 2 < N_SB_W)
        def _():
            in_desc(t1 + 2, 1).start()

        return c

    lax.fori_loop(1, (N_SB_W - 1) // 2, pair_body, 0)
    in_desc(N_SB_W - 1, 0).wait()
    out_desc(N_SB_W - 3, 0).wait()
    transpose_sb(0)
    out_desc(N_SB_W - 1, 0).start()
    out_desc(N_SB_W - 2, 1).wait()
    out_desc(N_SB_W - 1, 0).wait()

    @pl.when((wid >= 16) & (wid < 16 + N_LEFT))
    def _():
        c0 = SB_COVER + (wid - 16) * 128
        pltpu.sync_copy(tt.at[:, pl.ds(c0, 128)], vin_[0].at[:, pl.ds(0, 128)])

        def trl_body(m, c2):
            for jj in range(8):
                r = m * 8 + jj
                v = plsc.load_gather(vin_[0],
                                     [i16, jnp.full((16,), r, jnp.int32)])
                vout_[0][pl.ds(r * EMBED_DIM, EMBED_DIM)] = v
            return c2

        lax.fori_loop(0, 16, trl_body, 0)
        pltpu.sync_copy(vout_[0].at[pl.ds(0, 128 * EMBED_DIM)],
                        tab_lin.at[pl.ds(c0 * EMBED_DIM, 128 * EMBED_DIM)])

    @pl.when(wid == 20)
    def _():
        def te_body(e, c2):
            pltpu.sync_copy(tt.at[e, pl.ds(TAIL_START, TAIL_N)], tailbuf.at[e])
            return c2

        lax.fori_loop(0, EMBED_DIM, te_body, 0)

        def tr2_body(r, c2):
            v = plsc.load_gather(tailbuf, [i16, jnp.full((16,), r, jnp.int32)])
            tailout[pl.ds(r * EMBED_DIM, EMBED_DIM)] = v
            return c2

        lax.fori_loop(0, TAIL_N, tr2_body, 0)
        pltpu.sync_copy(tailout,
                        tab_lin.at[pl.ds(TAIL_START * EMBED_DIM,
                                         TAIL_N * EMBED_DIM)])


@functools.partial(
    pl.kernel,
    mesh=_mesh,
    out_type=jax.ShapeDtypeStruct((N_FIELDS, 2, BATCH * 8), jnp.float32),
    compiler_params=_gather_params,
    scratch_types=[
        pltpu.VMEM((BLK_PER_W * 128,), jnp.int32),
        pltpu.VMEM((CHUNK_BLKS * 128, EMBED_DIM), jnp.float32),
        pltpu.VMEM((CHUNK_BLKS * 128, EMBED_DIM), jnp.float32),
        pltpu.VMEM((2 * CHUNK_BLKS * 8 * 128,), jnp.float32),
        pltpu.VMEM((2 * CHUNK_BLKS * 8 * 128,), jnp.float32),
        pltpu.SemaphoreType.DMA,
        pltpu.SemaphoreType.DMA,
        pltpu.SemaphoreType.DMA,
        pltpu.SemaphoreType.DMA,
    ],
)
def _gather(tab2d, idx_lin, out5, idxall, rows0, rows1, outb0, outb1,
            gsem0, gsem1, wsem0, wsem1):
    wid = lax.axis_index("s") * NC + lax.axis_index("c")
    i16 = _iota16()
    rows_ = (rows0, rows1)
    outb_ = (outb0, outb1)
    gsem_ = (gsem0, gsem1)
    wsem_ = (wsem0, wsem1)
    half = CHUNK_BLKS * 8 * 128
    perlane = (i16 // 8) * half + (i16 % 8) * 128

    base_g = wid * BLK_PER_W
    pltpu.sync_copy(idx_lin.at[pl.ds(base_g * 128, BLK_PER_W * 128)], idxall)

    def g_start(j, p):
        return pltpu.async_copy(
            tab2d.at[idxall.at[pl.ds(j * CHUNK_BLKS * 128, CHUNK_BLKS * 128)]],
            rows_[p], gsem_[p])

    cp = g_start(0, 0)
    prev_out = [None, None]
    for j in range(NCHUNK):
        p = j & 1
        nxt = g_start(j + 1, 1 - p) if j + 1 < NCHUNK else None
        cp.wait()
        if prev_out[p] is not None:
            prev_out[p][0].wait()
            prev_out[p][1].wait()
        g0 = base_g + CHUNK_BLKS * j
        f = g0 // (BATCH // 128)
        bt0 = g0 % (BATCH // 128)

        def blk_body(k, c2, p=p):
            def lk_body(bl, idxv):
                for u in range(4):
                    v = rows_[p][k * 128 + bl * 4 + u, :]
                    plsc.store_scatter(outb_[p], [idxv + u], v)
                return idxv + 4

            lax.fori_loop(0, 32, lk_body, perlane + k * (8 * 128))
            return c2

        lax.fori_loop(0, CHUNK_BLKS, blk_body, 0)
        o1 = pltpu.async_copy(outb_[p].at[pl.ds(0, half)],
                              out5.at[f, 0, pl.ds(bt0 * 1024, half)], wsem_[p])
        o2 = pltpu.async_copy(outb_[p].at[pl.ds(half, half)],
                              out5.at[f, 1, pl.ds(bt0 * 1024, half)], wsem_[p])
        prev_out[p] = (o1, o2)
        cp = nxt
    for p in (0, 1):
        if prev_out[p] is not None:
            prev_out[p][0].wait()
            prev_out[p][1].wait()


@jax.jit
def kernel(indices, table):
    tab_lin, idx_lin = _detile(table.T, indices.T)
    out5 = _gather(tab_lin.reshape(VOCAB, EMBED_DIM), idx_lin)
    out5 = out5.reshape(N_FIELDS, 2, BATCH // 128, 8, 128)
    return out5.transpose(2, 4, 0, 1, 3).reshape(BATCH, N_FIELDS, EMBED_DIM)
